# baseline (speedup 1.0000x reference)
import functools

import jax
import jax.numpy as jnp
from jax import lax
from jax.experimental import pallas as pl
from jax.experimental.pallas import tpu as pltpu
from jax.experimental.pallas import tpu_sc as plsc

N_CLS = 10000
N_CTX = 16
SUF = 60
SEQ = 77
D = 128
B = 1024
NV = 4

NW = 32
PER_TILE = NV * B // NW
NBUF = 8
TILES_PER_V = NW // NV


def _sc_body(cls_ids, pn, cn, sn, pp, cp, sp, pe, ce, se, ps, cs, ss,
             out, idx_v, sbuf, gsem, wsem):
  cid = lax.axis_index("c")
  sid = lax.axis_index("s")
  wid = sid * 2 + cid
  bbuf = sbuf.at[sid]
  v = wid // TILES_PER_V
  b0 = (wid % TILES_PER_V) * PER_TILE
  i0_tile = wid * PER_TILE

  pltpu.sync_copy(cls_ids.at[pl.ds(b0, PER_TILE)], idx_v.at[pl.ds(0, PER_TILE)])

  lanes = lax.iota(jnp.int32, 16)

  def run(pref, ctxt, suft):
    def start_gathers(iv, t):
      pltpu.async_copy(pref.at[iv], bbuf.at[t, pl.ds(0, 1), :], gsem.at[t])
      pltpu.async_copy(ctxt.at[iv], bbuf.at[t, pl.ds(1, N_CTX), :], gsem.at[t])
      pltpu.async_copy(suft.at[iv], bbuf.at[t, pl.ds(1 + N_CTX, SUF), :], gsem.at[t])

    def retire(i, t):
      pltpu.make_async_copy(
          pref.at[0], bbuf.at[t, pl.ds(0, 1), :], gsem.at[t]).wait()
      pltpu.make_async_copy(
          ctxt.at[0], bbuf.at[t, pl.ds(1, N_CTX), :], gsem.at[t]).wait()
      pltpu.make_async_copy(
          suft.at[0], bbuf.at[t, pl.ds(1 + N_CTX, SUF), :], gsem.at[t]).wait()
      pltpu.async_copy(
          bbuf.at[t, pl.ds(0, SEQ), :], out.at[i0_tile + i], wsem.at[t]).wait()

    def loop_body(i, carry):
      t = lax.rem(i, NBUF)

      @pl.when(i >= NBUF)
      def _():
        retire(i - NBUF, t)

      base = (i // 8) * 8
      ivec = idx_v[pl.ds(base, 16)]
      iv = jnp.sum(jnp.where(lanes == i - base, ivec, 0))
      start_gathers(iv, t)
      return carry

    lax.fori_loop(0, PER_TILE, loop_body, 0)

    def tail_body(i, carry):
      retire(i - NBUF, lax.rem(i, NBUF))
      return carry

    lax.fori_loop(PER_TILE, PER_TILE + NBUF, tail_body, 0)

  @pl.when(v == 0)
  def _():
    run(pn, cn, sn)

  @pl.when(v == 1)
  def _():
    run(pp, cp, sp)

  @pl.when(v == 2)
  def _():
    run(pe, ce, se)

  @pl.when(v == 3)
  def _():
    run(ps, cs, ss)


_gather_call = functools.partial(
    pl.kernel,
    mesh=plsc.VectorSubcoreMesh(core_axis_name="c", subcore_axis_name="s"),
    out_type=jax.ShapeDtypeStruct((NV * B, SEQ, D), jnp.float32),
    scratch_types=[
        pltpu.VMEM((PER_TILE + 16,), jnp.int32),
        pltpu.VMEM_SHARED((16, NBUF, 80, D), jnp.float32),
        pltpu.SemaphoreType.DMA((NBUF,)),
        pltpu.SemaphoreType.DMA((NBUF,)),
    ],
    compiler_params=pltpu.CompilerParams(needs_layout_passes=False),
)(_sc_body)


@jax.jit
def kernel(cls_id, ctx_pos, ctx_neg, ctx_evi, ctx_sub,
           prefix_pos, suffix_pos, prefix_neg, suffix_neg,
           prefix_evi, suffix_evi, prefix_sub, suffix_sub):
  cls32 = cls_id.astype(jnp.int32)
  return _gather_call(
      cls32,
      prefix_neg, ctx_neg, suffix_neg,
      prefix_pos, ctx_pos, suffix_pos,
      prefix_evi, ctx_evi, suffix_evi,
      prefix_sub, ctx_sub, suffix_sub,
  )


# speedup vs baseline: 7.8150x; 7.8150x over previous
import functools

import jax
import jax.numpy as jnp
from jax import lax
from jax.experimental import pallas as pl
from jax.experimental.pallas import tpu as pltpu
from jax.experimental.pallas import tpu_sc as plsc

N_CLS = 10000
N_CTX = 16
SUF = 60
SEQ = 77
D = 128
B = 1024
NV = 4

NW = 32
PER_TILE = NV * B // NW
NBUF = 4
TILES_PER_V = NW // NV


def _sc_body(cls_ids, pn, cn, sn, pp, cp, sp, pe, ce, se, ps, cs, ss,
             out, idx_v, sibuf, gbuf, gsem, wsem):
  cid = lax.axis_index("c")
  sid = lax.axis_index("s")
  wid = sid * 2 + cid
  v = wid // TILES_PER_V
  b0 = (wid % TILES_PER_V) * PER_TILE
  i0 = wid * PER_TILE

  pltpu.sync_copy(cls_ids.at[pl.ds(b0, PER_TILE)], idx_v)

  def run(pref, ctxt, suft):
    def compute_rows(s, t):
      in_ctx = jnp.logical_and(s >= 1, s < 1 + N_CTX)
      mul = jnp.where(in_ctx, jnp.int32(N_CTX), jnp.int32(1))
      off = jnp.where(
          s == 0, jnp.int32(0),
          jnp.where(in_ctx, s - 1, (s - (1 + N_CTX)) * N_CLS))
      for k in range(PER_TILE // 16):
        ivec = idx_v[pl.ds(k * 16, 16)]
        sibuf[t, pl.ds(k * 16, 16)] = ivec * mul + off

    def start_gather(s, t):
      idx = sibuf.at[t]

      @pl.when(s == 0)
      def _():
        pltpu.async_copy(pref.at[idx], gbuf.at[t], gsem.at[t])

      @pl.when(jnp.logical_and(s >= 1, s < 1 + N_CTX))
      def _():
        pltpu.async_copy(ctxt.at[idx], gbuf.at[t], gsem.at[t])

      @pl.when(s >= 1 + N_CTX)
      def _():
        pltpu.async_copy(suft.at[idx], gbuf.at[t], gsem.at[t])

    def loop_body(s, carry):
      @pl.when(jnp.logical_and(s >= 2 * NBUF, s - 2 * NBUF < SEQ))
      def _():
        t = lax.rem(s, NBUF)
        pltpu.make_async_copy(
            gbuf.at[t], out.at[s - 2 * NBUF, pl.ds(i0, PER_TILE), :],
            wsem.at[t]).wait()

      @pl.when(jnp.logical_and(s >= NBUF, s - NBUF < SEQ))
      def _():
        t = lax.rem(s, NBUF)
        pltpu.make_async_copy(
            pref.at[pl.ds(0, PER_TILE)], gbuf.at[t], gsem.at[t]).wait()
        pltpu.async_copy(
            gbuf.at[t], out.at[s - NBUF, pl.ds(i0, PER_TILE), :], wsem.at[t])

      @pl.when(s < SEQ)
      def _():
        t = lax.rem(s, NBUF)
        compute_rows(s, t)
        start_gather(s, t)

      return carry

    lax.fori_loop(0, SEQ + 2 * NBUF, loop_body, 0)

  @pl.when(v == 0)
  def _():
    run(pn, cn, sn)

  @pl.when(v == 1)
  def _():
    run(pp, cp, sp)

  @pl.when(v == 2)
  def _():
    run(pe, ce, se)

  @pl.when(v == 3)
  def _():
    run(ps, cs, ss)


_gather_call = functools.partial(
    pl.kernel,
    mesh=plsc.VectorSubcoreMesh(core_axis_name="c", subcore_axis_name="s"),
    out_type=jax.ShapeDtypeStruct((SEQ, NV * B, D), jnp.float32),
    scratch_types=[
        pltpu.VMEM((PER_TILE,), jnp.int32),
        pltpu.VMEM((NBUF, PER_TILE), jnp.int32),
        pltpu.VMEM((NBUF, PER_TILE, D), jnp.float32),
        pltpu.SemaphoreType.DMA((NBUF,)),
        pltpu.SemaphoreType.DMA((NBUF,)),
    ],
    compiler_params=pltpu.CompilerParams(needs_layout_passes=False),
)(_sc_body)


@jax.jit
def kernel(cls_id, ctx_pos, ctx_neg, ctx_evi, ctx_sub,
           prefix_pos, suffix_pos, prefix_neg, suffix_neg,
           prefix_evi, suffix_evi, prefix_sub, suffix_sub):
  cls32 = cls_id.astype(jnp.int32)

  def pre2d(p):
    return p.reshape(N_CLS, D)

  def ctx2d(c):
    return c.reshape(N_CLS * N_CTX, D)

  def suf2d(s):
    return jnp.transpose(s, (1, 0, 2)).reshape(SUF * N_CLS, D)

  out = _gather_call(
      cls32,
      pre2d(prefix_neg), ctx2d(ctx_neg), suf2d(suffix_neg),
      pre2d(prefix_pos), ctx2d(ctx_pos), suf2d(suffix_pos),
      pre2d(prefix_evi), ctx2d(ctx_evi), suf2d(suffix_evi),
      pre2d(prefix_sub), ctx2d(ctx_sub), suf2d(suffix_sub),
  )
  return jnp.transpose(out, (1, 0, 2))


# speedup vs baseline: 7.8271x; 1.0015x over previous
import functools

import jax
import jax.numpy as jnp
from jax import lax
from jax.experimental import pallas as pl
from jax.experimental.pallas import tpu as pltpu
from jax.experimental.pallas import tpu_sc as plsc

N_CLS = 10000
N_CTX = 16
SUF = 60
SEQ = 77
D = 128
B = 1024
NV = 4

NW = 32
PER_TILE = NV * B // NW
NBUF = 4
TILES_PER_V = NW // NV


def _sc_body(cls_ids, pn, cn, sn, pp, cp, sp, pe, ce, se, ps, cs, ss,
             out, idx_v, sibuf, gbuf, gsem, wsem):
  cid = lax.axis_index("c")
  sid = lax.axis_index("s")
  wid = sid * 2 + cid
  v = wid // TILES_PER_V
  b0 = (wid % TILES_PER_V) * PER_TILE
  i0 = wid * PER_TILE

  pltpu.sync_copy(cls_ids.at[pl.ds(b0, PER_TILE)], idx_v)

  def run(pref, ctxt, suft):
    def compute_rows(s, t):
      in_ctx = jnp.logical_and(s >= 1, s < 1 + N_CTX)
      mul = jnp.where(in_ctx, jnp.int32(N_CTX), jnp.int32(1))
      off = jnp.where(
          s == 0, jnp.int32(0),
          jnp.where(in_ctx, s - 1, (s - (1 + N_CTX)) * N_CLS))
      for k in range(PER_TILE // 16):
        ivec = idx_v[pl.ds(k * 16, 16)]
        sibuf[t, pl.ds(k * 16, 16)] = ivec * mul + off

    def start_gather(s, t):
      idx = sibuf.at[t]

      @pl.when(s == 0)
      def _():
        pltpu.async_copy(pref.at[idx], gbuf.at[t], gsem.at[t])

      @pl.when(jnp.logical_and(s >= 1, s < 1 + N_CTX))
      def _():
        pltpu.async_copy(ctxt.at[idx], gbuf.at[t], gsem.at[t])

      @pl.when(s >= 1 + N_CTX)
      def _():
        pltpu.async_copy(suft.at[idx], gbuf.at[t], gsem.at[t])

    WLAG = NBUF // 2

    def loop_body(s, carry):
      @pl.when(jnp.logical_and(s >= NBUF, s - NBUF < SEQ))
      def _():
        t = lax.rem(s, NBUF)
        pltpu.make_async_copy(
            gbuf.at[t], out.at[s - NBUF, pl.ds(i0, PER_TILE), :],
            wsem.at[t]).wait()

      @pl.when(jnp.logical_and(s >= WLAG, s - WLAG < SEQ))
      def _():
        t = lax.rem(s - WLAG, NBUF)
        pltpu.make_async_copy(
            pref.at[pl.ds(0, PER_TILE)], gbuf.at[t], gsem.at[t]).wait()
        pltpu.async_copy(
            gbuf.at[t], out.at[s - WLAG, pl.ds(i0, PER_TILE), :], wsem.at[t])

      @pl.when(s < SEQ)
      def _():
        t = lax.rem(s, NBUF)
        compute_rows(s, t)
        start_gather(s, t)

      return carry

    lax.fori_loop(0, SEQ + NBUF, loop_body, 0)

  @pl.when(v == 0)
  def _():
    run(pn, cn, sn)

  @pl.when(v == 1)
  def _():
    run(pp, cp, sp)

  @pl.when(v == 2)
  def _():
    run(pe, ce, se)

  @pl.when(v == 3)
  def _():
    run(ps, cs, ss)


_gather_call = functools.partial(
    pl.kernel,
    mesh=plsc.VectorSubcoreMesh(core_axis_name="c", subcore_axis_name="s"),
    out_type=jax.ShapeDtypeStruct((SEQ, NV * B, D), jnp.float32),
    scratch_types=[
        pltpu.VMEM((PER_TILE,), jnp.int32),
        pltpu.VMEM((NBUF, PER_TILE), jnp.int32),
        pltpu.VMEM((NBUF, PER_TILE, D), jnp.float32),
        pltpu.SemaphoreType.DMA((NBUF,)),
        pltpu.SemaphoreType.DMA((NBUF,)),
    ],
    compiler_params=pltpu.CompilerParams(needs_layout_passes=False),
)(_sc_body)


@jax.jit
def kernel(cls_id, ctx_pos, ctx_neg, ctx_evi, ctx_sub,
           prefix_pos, suffix_pos, prefix_neg, suffix_neg,
           prefix_evi, suffix_evi, prefix_sub, suffix_sub):
  cls32 = cls_id.astype(jnp.int32)

  def pre2d(p):
    return p.reshape(N_CLS, D)

  def ctx2d(c):
    return c.reshape(N_CLS * N_CTX, D)

  def suf2d(s):
    return jnp.transpose(s, (1, 0, 2)).reshape(SUF * N_CLS, D)

  out = _gather_call(
      cls32,
      pre2d(prefix_neg), ctx2d(ctx_neg), suf2d(suffix_neg),
      pre2d(prefix_pos), ctx2d(ctx_pos), suf2d(suffix_pos),
      pre2d(prefix_evi), ctx2d(ctx_evi), suf2d(suffix_evi),
      pre2d(prefix_sub), ctx2d(ctx_sub), suf2d(suffix_sub),
  )
  return jnp.transpose(out, (1, 0, 2))


# speedup vs baseline: 7.8404x; 1.0017x over previous
import functools

import jax
import jax.numpy as jnp
from jax import lax
from jax.experimental import pallas as pl
from jax.experimental.pallas import tpu as pltpu
from jax.experimental.pallas import tpu_sc as plsc

N_CLS = 10000
N_CTX = 16
SUF = 60
SEQ = 77
D = 128
B = 1024
NV = 4

NW = 32
PER_TILE = NV * B // NW
NBUF = 6
TILES_PER_V = NW // NV


def _sc_body(cls_ids, pn, cn, sn, pp, cp, sp, pe, ce, se, ps, cs, ss,
             out, idx_v, sibuf, gbuf, gsem, wsem):
  cid = lax.axis_index("c")
  sid = lax.axis_index("s")
  wid = sid * 2 + cid
  v = wid // TILES_PER_V
  b0 = (wid % TILES_PER_V) * PER_TILE
  i0 = wid * PER_TILE

  pltpu.sync_copy(cls_ids.at[pl.ds(b0, PER_TILE)], idx_v)

  def run(pref, ctxt, suft):
    def compute_rows(s, t):
      in_ctx = jnp.logical_and(s >= 1, s < 1 + N_CTX)
      mul = jnp.where(in_ctx, jnp.int32(N_CTX), jnp.int32(1))
      off = jnp.where(
          s == 0, jnp.int32(0),
          jnp.where(in_ctx, s - 1, (s - (1 + N_CTX)) * N_CLS))
      for k in range(PER_TILE // 16):
        ivec = idx_v[pl.ds(k * 16, 16)]
        sibuf[t, pl.ds(k * 16, 16)] = ivec * mul + off

    def start_gather(s, t):
      idx = sibuf.at[t]

      @pl.when(s == 0)
      def _():
        pltpu.async_copy(pref.at[idx], gbuf.at[t], gsem.at[t])

      @pl.when(jnp.logical_and(s >= 1, s < 1 + N_CTX))
      def _():
        pltpu.async_copy(ctxt.at[idx], gbuf.at[t], gsem.at[t])

      @pl.when(s >= 1 + N_CTX)
      def _():
        pltpu.async_copy(suft.at[idx], gbuf.at[t], gsem.at[t])

    WLAG = NBUF // 2

    def loop_body(s, carry):
      @pl.when(jnp.logical_and(s >= NBUF, s - NBUF < SEQ))
      def _():
        t = lax.rem(s, NBUF)
        pltpu.make_async_copy(
            gbuf.at[t], out.at[s - NBUF, pl.ds(i0, PER_TILE), :],
            wsem.at[t]).wait()

      @pl.when(jnp.logical_and(s >= WLAG, s - WLAG < SEQ))
      def _():
        t = lax.rem(s - WLAG, NBUF)
        pltpu.make_async_copy(
            pref.at[pl.ds(0, PER_TILE)], gbuf.at[t], gsem.at[t]).wait()
        pltpu.async_copy(
            gbuf.at[t], out.at[s - WLAG, pl.ds(i0, PER_TILE), :], wsem.at[t])

      @pl.when(s < SEQ)
      def _():
        t = lax.rem(s, NBUF)
        compute_rows(s, t)
        start_gather(s, t)

      return carry

    lax.fori_loop(0, SEQ + NBUF, loop_body, 0)

  @pl.when(v == 0)
  def _():
    run(pn, cn, sn)

  @pl.when(v == 1)
  def _():
    run(pp, cp, sp)

  @pl.when(v == 2)
  def _():
    run(pe, ce, se)

  @pl.when(v == 3)
  def _():
    run(ps, cs, ss)


_gather_call = functools.partial(
    pl.kernel,
    mesh=plsc.VectorSubcoreMesh(core_axis_name="c", subcore_axis_name="s"),
    out_type=jax.ShapeDtypeStruct((SEQ, NV * B, D), jnp.float32),
    scratch_types=[
        pltpu.VMEM((PER_TILE,), jnp.int32),
        pltpu.VMEM((NBUF, PER_TILE), jnp.int32),
        pltpu.VMEM((NBUF, PER_TILE, D), jnp.float32),
        pltpu.SemaphoreType.DMA((NBUF,)),
        pltpu.SemaphoreType.DMA((NBUF,)),
    ],
    compiler_params=pltpu.CompilerParams(needs_layout_passes=False),
)(_sc_body)


@jax.jit
def kernel(cls_id, ctx_pos, ctx_neg, ctx_evi, ctx_sub,
           prefix_pos, suffix_pos, prefix_neg, suffix_neg,
           prefix_evi, suffix_evi, prefix_sub, suffix_sub):
  cls32 = cls_id.astype(jnp.int32)

  def pre2d(p):
    return p.reshape(N_CLS, D)

  def ctx2d(c):
    return c.reshape(N_CLS * N_CTX, D)

  def suf2d(s):
    return jnp.transpose(s, (1, 0, 2)).reshape(SUF * N_CLS, D)

  out = _gather_call(
      cls32,
      pre2d(prefix_neg), ctx2d(ctx_neg), suf2d(suffix_neg),
      pre2d(prefix_pos), ctx2d(ctx_pos), suf2d(suffix_pos),
      pre2d(prefix_evi), ctx2d(ctx_evi), suf2d(suffix_evi),
      pre2d(prefix_sub), ctx2d(ctx_sub), suf2d(suffix_sub),
  )
  return jnp.transpose(out, (1, 0, 2))
